# baseline (speedup 1.0000x reference)
import functools

import jax
import jax.numpy as jnp
from jax import lax
from jax.experimental import pallas as pl
from jax.experimental.pallas import tpu as pltpu
from jax.experimental.pallas import tpu_sc as plsc

N_NODES = 10000
D_FEAT = 128
NNZ = 320000

NC = 2
NS = 16
NW = NC * NS
K = 64
E_T = 10240
NNZ_PAD = NW * E_T
N_CHUNKS = E_T // K
NB = 5
DEPTH = 3
ROWS_T = 624
REM_ROWS = N_NODES - NS * ROWS_T

_mesh = plsc.VectorSubcoreMesh(core_axis_name="c", subcore_axis_name="s")


@functools.partial(
    pl.kernel,
    out_type=(
        jax.ShapeDtypeStruct((N_NODES, D_FEAT), jnp.float32),
        jax.ShapeDtypeStruct((N_NODES, D_FEAT), jnp.float32),
    ),
    mesh=_mesh,
    compiler_params=pltpu.CompilerParams(needs_layout_passes=False),
    scratch_types=[
        pltpu.VMEM_SHARED((N_NODES, D_FEAT), jnp.float32),
        pltpu.VMEM((NB, K, D_FEAT), jnp.float32),
        [pltpu.VMEM((K,), jnp.int32) for _ in range(NB)],
        pltpu.VMEM((NB, K), jnp.int32),
        pltpu.VMEM((NB, K), jnp.float32),
        pltpu.SemaphoreType.DMA((NB,)),
        pltpu.SemaphoreType.DMA((NB,)),
    ],
)
def _spmm(x_hbm, rows_hbm, cols_hbm, vals_hbm, zeros_hbm, out0, out1,
          acc, gath6, rowb, col6, val6, sg, si):
    c = lax.axis_index("c")
    s = lax.axis_index("s")
    wid = c * NS + s

    def base_of(i):
        return wid * E_T + i * K

    def start_idx(i, b):
        base = base_of(i)
        pltpu.async_copy(rows_hbm.at[pl.ds(base, K)], rowb[b], si.at[b])
        pltpu.async_copy(cols_hbm.at[pl.ds(base, K)], col6.at[b], si.at[b])
        pltpu.async_copy(vals_hbm.at[pl.ds(base, K)], val6.at[b], si.at[b])

    def wait_idx(i, b):
        base = base_of(i)
        pltpu.make_async_copy(rows_hbm.at[pl.ds(base, K)], rowb[b], si.at[b]).wait()
        pltpu.make_async_copy(cols_hbm.at[pl.ds(base, K)], col6.at[b], si.at[b]).wait()
        pltpu.make_async_copy(vals_hbm.at[pl.ds(base, K)], val6.at[b], si.at[b]).wait()

    def start_gather(b):
        pltpu.async_copy(x_hbm.at[col6.at[b]], gath6.at[b], sg.at[b])

    def wait_gather(b):
        pltpu.make_async_copy(x_hbm.at[col6.at[b]], gath6.at[b], sg.at[b]).wait()

    with jax.named_scope("zph"):
        zsl = pl.ds(s * ROWS_T, ROWS_T)
        pltpu.sync_copy(zeros_hbm.at[zsl], acc.at[zsl])

        @pl.when(s == NS - 1)
        def _():
            zrem = pl.ds(NS * ROWS_T, REM_ROWS)
            pltpu.sync_copy(zeros_hbm.at[zrem], acc.at[zrem])

    with jax.named_scope("zbar"):
        plsc.subcore_barrier()

    def scale_chunk(gref, vref):
        def grp(g, carry2):
            for jj in range(16):
                j = g * 16 + jj
                sp = plsc.load_gather(vref, [jnp.full((16,), j, jnp.int32)])
                r = gref.at[j]
                for f in range(D_FEAT // 16):
                    r[pl.ds(f * 16, 16)] = r[pl.ds(f * 16, 16)] * sp
            return carry2

        lax.fori_loop(0, K // 16, grp, 0)

    def do_chunk(i, b):
        with jax.named_scope("gwait"):
            wait_gather(b)

        with jax.named_scope("iwait"):
            @pl.when(i + DEPTH < N_CHUNKS)
            def _():
                bg = (b + DEPTH) % NB
                wait_idx(i + DEPTH, bg)
                start_gather(bg)

            @pl.when(i + DEPTH + 1 < N_CHUNKS)
            def _():
                start_idx(i + DEPTH + 1, (b + DEPTH + 1) % NB)

        with jax.named_scope("scale"):
            scale_chunk(gath6.at[b], val6.at[b])
        with jax.named_scope("scat"):
            pltpu.sync_copy(gath6.at[b], acc.at[rowb[b]], add=True)

    for j in range(DEPTH + 1):
        start_idx(j, j)
    for j in range(DEPTH):
        wait_idx(j, j)
        start_gather(j)

    MAIN = (N_CHUNKS // NB) * NB

    def six(k, carry):
        i0 = k * NB
        for off in range(NB):
            do_chunk(i0 + off, off)
        return carry

    lax.fori_loop(0, MAIN // NB, six, 0)
    for i in range(MAIN, N_CHUNKS):
        do_chunk(i, i % NB)
    with jax.named_scope("ebar"):
        plsc.subcore_barrier()

    sl = pl.ds(s * ROWS_T, ROWS_T)
    rem = pl.ds(NS * ROWS_T, REM_ROWS)

    with jax.named_scope("wph"):
        @pl.when(c == 0)
        def _():
            pltpu.sync_copy(acc.at[sl], out0.at[sl])

            @pl.when(s == NS - 1)
            def _():
                pltpu.sync_copy(acc.at[rem], out0.at[rem])

        @pl.when(c == 1)
        def _():
            pltpu.sync_copy(acc.at[sl], out1.at[sl])

            @pl.when(s == NS - 1)
            def _():
                pltpu.sync_copy(acc.at[rem], out1.at[rem])


def _ew_call(body, n_out):
    out = tuple(jax.ShapeDtypeStruct((N_NODES, D_FEAT), jnp.float32)
                for _ in range(n_out))
    return pl.pallas_call(body, out_shape=out[0] if n_out == 1 else out)


def _add2_body(a, b, o):
    o[...] = a[...] + b[...]


def _resid_body(q0, q1, xp, tp, xo, to):
    x = q0[...] + q1[...] + xp[...]
    xo[...] = x
    to[...] = tp[...] + x


def _final_body(q0, q1, xp, tp, o):
    o[...] = (tp[...] + q0[...] + q1[...] + xp[...]) * 0.25


_add2 = _ew_call(_add2_body, 1)
_resid = _ew_call(_resid_body, 2)
_final = _ew_call(_final_body, 1)


def _prep(indices, values):
    idx = indices.astype(jnp.int32)
    pad = NNZ_PAD - NNZ
    rows = jnp.concatenate([idx[0], jnp.zeros((pad,), jnp.int32)])
    cols = jnp.concatenate([idx[1], jnp.zeros((pad,), jnp.int32)])
    vals = jnp.concatenate([values.astype(jnp.float32),
                            jnp.zeros((pad,), jnp.float32)])
    return rows, cols, vals


def kernel(poi_embs, src_indices, src_values, tar_indices, tar_values):
    tr, tcol, tval = _prep(tar_indices, tar_values)
    sr, scol, sval = _prep(src_indices, src_values)
    x = poi_embs
    t = poi_embs
    out = None
    zeros = jnp.zeros((N_NODES, D_FEAT), jnp.float32)
    for layer in range(3):
        p0, p1 = _spmm(x, tr, tcol, tval, zeros)
        m = _add2(p0, p1)
        q0, q1 = _spmm(m, sr, scol, sval, zeros)
        if layer < 2:
            x, t = _resid(q0, q1, x, t)
        else:
            out = _final(q0, q1, x, t)
    return out


# speedup vs baseline: 1.3312x; 1.3312x over previous
import functools

import jax
import jax.numpy as jnp
from jax import lax
from jax.experimental import pallas as pl
from jax.experimental.pallas import tpu as pltpu
from jax.experimental.pallas import tpu_sc as plsc

N_NODES = 10000
D_FEAT = 128
NNZ = 320000

NC = 2
NS = 16
NW = NC * NS
K = 64
N_CH0 = 215
N_CH1 = 105
E_T0 = N_CH0 * K
E_T1 = N_CH1 * K
NNZ_PAD = NS * (E_T0 + E_T1)
NB = 5
DEPTH = 3
ROWS_T = 624
REM_ROWS = N_NODES - NS * ROWS_T

_mesh = plsc.VectorSubcoreMesh(core_axis_name="c", subcore_axis_name="s")


@functools.partial(
    pl.kernel,
    out_type=(
        jax.ShapeDtypeStruct((N_NODES, D_FEAT), jnp.float32),
        jax.ShapeDtypeStruct((N_NODES, D_FEAT), jnp.float32),
    ),
    mesh=_mesh,
    compiler_params=pltpu.CompilerParams(needs_layout_passes=False),
    scratch_types=[
        pltpu.VMEM_SHARED((N_NODES, D_FEAT), jnp.float32),
        pltpu.VMEM((NB, K, D_FEAT), jnp.float32),
        [pltpu.VMEM((K,), jnp.int32) for _ in range(NB)],
        pltpu.VMEM((NB, K), jnp.int32),
        pltpu.VMEM((NB, K), jnp.float32),
        pltpu.SemaphoreType.DMA((NB,)),
        pltpu.SemaphoreType.DMA((NB,)),
    ],
)
def _spmm(x_hbm, rows_hbm, cols_hbm, vals_hbm, zeros_hbm, out0, out1,
          acc, gath6, rowb, col6, val6, sg, si):
    c = lax.axis_index("c")
    s = lax.axis_index("s")
    n_chunks = jnp.where(c == 0, N_CH0, N_CH1).astype(jnp.int32)
    tile_base = jnp.where(c == 0, s * E_T0, NS * E_T0 + s * E_T1)

    def base_of(i):
        return tile_base + i * K

    def start_idx(i, b):
        base = base_of(i)
        pltpu.async_copy(rows_hbm.at[pl.ds(base, K)], rowb[b], si.at[b])
        pltpu.async_copy(cols_hbm.at[pl.ds(base, K)], col6.at[b], si.at[b])
        pltpu.async_copy(vals_hbm.at[pl.ds(base, K)], val6.at[b], si.at[b])

    def wait_idx(i, b):
        base = base_of(i)
        pltpu.make_async_copy(rows_hbm.at[pl.ds(base, K)], rowb[b], si.at[b]).wait()
        pltpu.make_async_copy(cols_hbm.at[pl.ds(base, K)], col6.at[b], si.at[b]).wait()
        pltpu.make_async_copy(vals_hbm.at[pl.ds(base, K)], val6.at[b], si.at[b]).wait()

    def start_gather(b):
        pltpu.async_copy(x_hbm.at[col6.at[b]], gath6.at[b], sg.at[b])

    def wait_gather(b):
        pltpu.make_async_copy(x_hbm.at[col6.at[b]], gath6.at[b], sg.at[b]).wait()

    with jax.named_scope("zph"):
        zsl = pl.ds(s * ROWS_T, ROWS_T)
        pltpu.sync_copy(zeros_hbm.at[zsl], acc.at[zsl])

        @pl.when(s == NS - 1)
        def _():
            zrem = pl.ds(NS * ROWS_T, REM_ROWS)
            pltpu.sync_copy(zeros_hbm.at[zrem], acc.at[zrem])

    with jax.named_scope("zbar"):
        plsc.subcore_barrier()

    def scale_chunk(gref, vref):
        def grp(g, carry2):
            for jj in range(16):
                j = g * 16 + jj
                sp = plsc.load_gather(vref, [jnp.full((16,), j, jnp.int32)])
                r = gref.at[j]
                for f in range(D_FEAT // 16):
                    r[pl.ds(f * 16, 16)] = r[pl.ds(f * 16, 16)] * sp
            return carry2

        lax.fori_loop(0, K // 16, grp, 0)

    def do_chunk(i, b):
        with jax.named_scope("gwait"):
            wait_gather(b)

        with jax.named_scope("iwait"):
            @pl.when(i + DEPTH < n_chunks)
            def _():
                bg = (b + DEPTH) % NB
                wait_idx(i + DEPTH, bg)
                start_gather(bg)

            @pl.when(i + DEPTH + 1 < n_chunks)
            def _():
                start_idx(i + DEPTH + 1, (b + DEPTH + 1) % NB)

        with jax.named_scope("scale"):
            scale_chunk(gath6.at[b], val6.at[b])
        with jax.named_scope("scat"):
            pltpu.sync_copy(gath6.at[b], acc.at[rowb[b]], add=True)

    for j in range(DEPTH + 1):
        start_idx(j, j)
    for j in range(DEPTH):
        wait_idx(j, j)
        start_gather(j)

    def five(k, carry):
        i0 = k * NB
        for off in range(NB):
            do_chunk(i0 + off, off)
        return carry

    lax.fori_loop(0, n_chunks // NB, five, 0)
    with jax.named_scope("ebar"):
        plsc.subcore_barrier()

    sl = pl.ds(s * ROWS_T, ROWS_T)
    rem = pl.ds(NS * ROWS_T, REM_ROWS)

    with jax.named_scope("wph"):
        @pl.when(c == 0)
        def _():
            pltpu.sync_copy(acc.at[sl], out0.at[sl])

            @pl.when(s == NS - 1)
            def _():
                pltpu.sync_copy(acc.at[rem], out0.at[rem])

        @pl.when(c == 1)
        def _():
            pltpu.sync_copy(acc.at[sl], out1.at[sl])

            @pl.when(s == NS - 1)
            def _():
                pltpu.sync_copy(acc.at[rem], out1.at[rem])


def _ew_call(body, n_out):
    out = tuple(jax.ShapeDtypeStruct((N_NODES, D_FEAT), jnp.float32)
                for _ in range(n_out))
    return pl.pallas_call(body, out_shape=out[0] if n_out == 1 else out)


def _add2_body(a, b, o):
    o[...] = a[...] + b[...]


def _resid_body(q0, q1, xp, tp, xo, to):
    x = q0[...] + q1[...] + xp[...]
    xo[...] = x
    to[...] = tp[...] + x


def _final_body(q0, q1, xp, tp, o):
    o[...] = (tp[...] + q0[...] + q1[...] + xp[...]) * 0.25


_add2 = _ew_call(_add2_body, 1)
_resid = _ew_call(_resid_body, 2)
_final = _ew_call(_final_body, 1)


def _prep(indices, values):
    idx = indices.astype(jnp.int32)
    pad = NNZ_PAD - NNZ
    rows = jnp.concatenate([idx[0], jnp.zeros((pad,), jnp.int32)])
    cols = jnp.concatenate([idx[1], jnp.zeros((pad,), jnp.int32)])
    vals = jnp.concatenate([values.astype(jnp.float32),
                            jnp.zeros((pad,), jnp.float32)])
    return rows, cols, vals


def kernel(poi_embs, src_indices, src_values, tar_indices, tar_values):
    tr, tcol, tval = _prep(tar_indices, tar_values)
    sr, scol, sval = _prep(src_indices, src_values)
    x = poi_embs
    t = poi_embs
    out = None
    zeros = jnp.zeros((N_NODES, D_FEAT), jnp.float32)
    for layer in range(3):
        p0, p1 = _spmm(x, tr, tcol, tval, zeros)
        m = _add2(p0, p1)
        q0, q1 = _spmm(m, sr, scol, sval, zeros)
        if layer < 2:
            x, t = _resid(q0, q1, x, t)
        else:
            out = _final(q0, q1, x, t)
    return out


# speedup vs baseline: 1.3632x; 1.0240x over previous
import functools

import jax
import jax.numpy as jnp
from jax import lax
from jax.experimental import pallas as pl
from jax.experimental.pallas import tpu as pltpu
from jax.experimental.pallas import tpu_sc as plsc

N_NODES = 10000
D_FEAT = 128
NNZ = 320000

NC = 2
NS = 16
NW = NC * NS
K = 64
N_CH0 = 180
N_CH1 = 140
E_T0 = N_CH0 * K
E_T1 = N_CH1 * K
NNZ_PAD = NS * (E_T0 + E_T1)
NB = 5
DEPTH = 3
ROWS_T = 624
REM_ROWS = N_NODES - NS * ROWS_T

_mesh = plsc.VectorSubcoreMesh(core_axis_name="c", subcore_axis_name="s")


@functools.partial(
    pl.kernel,
    out_type=(
        jax.ShapeDtypeStruct((N_NODES, D_FEAT), jnp.float32),
        jax.ShapeDtypeStruct((N_NODES, D_FEAT), jnp.float32),
    ),
    mesh=_mesh,
    compiler_params=pltpu.CompilerParams(needs_layout_passes=False,
                                         use_tc_tiling_on_sc=False),
    scratch_types=[
        pltpu.VMEM_SHARED((N_NODES, D_FEAT), jnp.float32),
        pltpu.VMEM((NB, K, D_FEAT), jnp.bfloat16),
        pltpu.VMEM((K, D_FEAT), jnp.float32),
        [pltpu.VMEM((K,), jnp.int32) for _ in range(NB)],
        pltpu.VMEM((NB, K), jnp.int32),
        pltpu.VMEM((NB, K), jnp.float32),
        pltpu.SemaphoreType.DMA((NB,)),
        pltpu.SemaphoreType.DMA((NB,)),
    ],
)
def _spmm(x_hbm, rows_hbm, cols_hbm, vals_hbm, zeros_hbm, out0, out1,
          acc, gath6, gathf, rowb, col6, val6, sg, si):
    c = lax.axis_index("c")
    s = lax.axis_index("s")
    n_chunks = jnp.where(c == 0, N_CH0, N_CH1).astype(jnp.int32)
    tile_base = jnp.where(c == 0, s * E_T0, NS * E_T0 + s * E_T1)

    def base_of(i):
        return tile_base + i * K

    def start_idx(i, b):
        base = base_of(i)
        pltpu.async_copy(rows_hbm.at[pl.ds(base, K)], rowb[b], si.at[b])
        pltpu.async_copy(cols_hbm.at[pl.ds(base, K)], col6.at[b], si.at[b])
        pltpu.async_copy(vals_hbm.at[pl.ds(base, K)], val6.at[b], si.at[b])

    def wait_idx(i, b):
        base = base_of(i)
        pltpu.make_async_copy(rows_hbm.at[pl.ds(base, K)], rowb[b], si.at[b]).wait()
        pltpu.make_async_copy(cols_hbm.at[pl.ds(base, K)], col6.at[b], si.at[b]).wait()
        pltpu.make_async_copy(vals_hbm.at[pl.ds(base, K)], val6.at[b], si.at[b]).wait()

    def start_gather(b):
        pltpu.async_copy(x_hbm.at[col6.at[b]], gath6.at[b], sg.at[b])

    def wait_gather(b):
        pltpu.make_async_copy(x_hbm.at[col6.at[b]], gath6.at[b], sg.at[b]).wait()

    with jax.named_scope("zph"):
        zsl = pl.ds(s * ROWS_T, ROWS_T)
        pltpu.sync_copy(zeros_hbm.at[zsl], acc.at[zsl])

        @pl.when(s == NS - 1)
        def _():
            zrem = pl.ds(NS * ROWS_T, REM_ROWS)
            pltpu.sync_copy(zeros_hbm.at[zrem], acc.at[zrem])

    with jax.named_scope("zbar"):
        plsc.subcore_barrier()

    def scale_chunk(gref, vref):
        def grp(g, carry2):
            for jj in range(16):
                j = g * 16 + jj
                sp = plsc.load_gather(vref, [jnp.full((16,), j, jnp.int32)])
                r = gref.at[j]
                o = gathf.at[j]
                for f in range(D_FEAT // 32):
                    ab = r[pl.ds(f * 32, 32)]
                    a, b2 = plsc.unpack(ab, format=plsc.PackFormat.INTERLEAVED,
                                        preferred_element_type=jnp.float32)
                    o[pl.ds(f * 32, 16)] = a * sp
                    o[pl.ds(f * 32 + 16, 16)] = b2 * sp
            return carry2

        lax.fori_loop(0, K // 16, grp, 0)

    def do_chunk(i, b):
        with jax.named_scope("gwait"):
            wait_gather(b)

        with jax.named_scope("iwait"):
            @pl.when(i + DEPTH < n_chunks)
            def _():
                bg = (b + DEPTH) % NB
                wait_idx(i + DEPTH, bg)
                start_gather(bg)

            @pl.when(i + DEPTH + 1 < n_chunks)
            def _():
                start_idx(i + DEPTH + 1, (b + DEPTH + 1) % NB)

        with jax.named_scope("scale"):
            scale_chunk(gath6.at[b], val6.at[b])
        with jax.named_scope("scat"):
            pltpu.sync_copy(gathf, acc.at[rowb[b]], add=True)

    for j in range(DEPTH + 1):
        start_idx(j, j)
    for j in range(DEPTH):
        wait_idx(j, j)
        start_gather(j)

    def five(k, carry):
        i0 = k * NB
        for off in range(NB):
            do_chunk(i0 + off, off)
        return carry

    lax.fori_loop(0, n_chunks // NB, five, 0)
    with jax.named_scope("ebar"):
        plsc.subcore_barrier()

    sl = pl.ds(s * ROWS_T, ROWS_T)
    rem = pl.ds(NS * ROWS_T, REM_ROWS)

    with jax.named_scope("wph"):
        @pl.when(c == 0)
        def _():
            pltpu.sync_copy(acc.at[sl], out0.at[sl])

            @pl.when(s == NS - 1)
            def _():
                pltpu.sync_copy(acc.at[rem], out0.at[rem])

        @pl.when(c == 1)
        def _():
            pltpu.sync_copy(acc.at[sl], out1.at[sl])

            @pl.when(s == NS - 1)
            def _():
                pltpu.sync_copy(acc.at[rem], out1.at[rem])


def _ew_call(body, n_out):
    out = tuple(jax.ShapeDtypeStruct((N_NODES, D_FEAT), jnp.float32)
                for _ in range(n_out))
    return pl.pallas_call(body, out_shape=out[0] if n_out == 1 else out)


def _add2_body(a, b, o):
    o[...] = a[...] + b[...]


def _resid_body(q0, q1, xp, tp, xo, to):
    x = q0[...] + q1[...] + xp[...]
    xo[...] = x
    to[...] = tp[...] + x


def _final_body(q0, q1, xp, tp, o):
    o[...] = (tp[...] + q0[...] + q1[...] + xp[...]) * 0.25


_add2 = _ew_call(_add2_body, 1)
_resid = _ew_call(_resid_body, 2)
_final = _ew_call(_final_body, 1)


def _to_bf16i(x):
    xi = x.reshape(N_NODES, D_FEAT // 32, 2, 16).swapaxes(2, 3)
    return xi.reshape(N_NODES, D_FEAT).astype(jnp.bfloat16)


def _prep(indices, values):
    idx = indices.astype(jnp.int32)
    pad = NNZ_PAD - NNZ
    rows = jnp.concatenate([idx[0], jnp.zeros((pad,), jnp.int32)])
    cols = jnp.concatenate([idx[1], jnp.zeros((pad,), jnp.int32)])
    vals = jnp.concatenate([values.astype(jnp.float32),
                            jnp.zeros((pad,), jnp.float32)])
    return rows, cols, vals


def kernel(poi_embs, src_indices, src_values, tar_indices, tar_values):
    tr, tcol, tval = _prep(tar_indices, tar_values)
    sr, scol, sval = _prep(src_indices, src_values)
    x = poi_embs
    t = poi_embs
    out = None
    zeros = jnp.zeros((N_NODES, D_FEAT), jnp.float32)
    for layer in range(3):
        p0, p1 = _spmm(_to_bf16i(x), tr, tcol, tval, zeros)
        m = _add2(p0, p1)
        q0, q1 = _spmm(_to_bf16i(m), sr, scol, sval, zeros)
        if layer < 2:
            x, t = _resid(q0, q1, x, t)
        else:
            out = _final(q0, q1, x, t)
    return out


# speedup vs baseline: 1.3814x; 1.0134x over previous
import functools

import jax
import jax.numpy as jnp
from jax import lax
from jax.experimental import pallas as pl
from jax.experimental.pallas import tpu as pltpu
from jax.experimental.pallas import tpu_sc as plsc

N_NODES = 10000
D_FEAT = 128
NNZ = 320000

NC = 2
NS = 16
NW = NC * NS
K = 64
N_CH0 = 180
N_CH1 = 140
E_T0 = N_CH0 * K
E_T1 = N_CH1 * K
NNZ_PAD = NS * (E_T0 + E_T1)
NB = 5
DEPTH = 3
ROWS_T = 624
REM_ROWS = N_NODES - NS * ROWS_T

_mesh = plsc.VectorSubcoreMesh(core_axis_name="c", subcore_axis_name="s")


@functools.partial(
    pl.kernel,
    out_type=(
        jax.ShapeDtypeStruct((N_NODES, D_FEAT), jnp.float32),
        jax.ShapeDtypeStruct((N_NODES, D_FEAT), jnp.float32),
    ),
    mesh=_mesh,
    compiler_params=pltpu.CompilerParams(needs_layout_passes=False,
                                         use_tc_tiling_on_sc=False),
    scratch_types=[
        pltpu.VMEM_SHARED((N_NODES, D_FEAT), jnp.float32),
        pltpu.VMEM((NB, K, D_FEAT // 2), jnp.int32),
        pltpu.VMEM((K, D_FEAT), jnp.float32),
        [pltpu.VMEM((K,), jnp.int32) for _ in range(NB)],
        pltpu.VMEM((NB, K), jnp.int32),
        pltpu.VMEM((NB, K), jnp.float32),
        pltpu.SemaphoreType.DMA((NB,)),
        pltpu.SemaphoreType.DMA((NB,)),
    ],
)
def _spmm(x_hbm, rows_hbm, cols_hbm, vals_hbm, zeros_hbm, out0, out1,
          acc, gath6, gathf, rowb, col6, val6, sg, si):
    c = lax.axis_index("c")
    s = lax.axis_index("s")
    n_chunks = jnp.where(c == 0, N_CH0, N_CH1).astype(jnp.int32)
    tile_base = jnp.where(c == 0, s * E_T0, NS * E_T0 + s * E_T1)

    def base_of(i):
        return tile_base + i * K

    def start_idx(i, b):
        base = base_of(i)
        pltpu.async_copy(rows_hbm.at[pl.ds(base, K)], rowb[b], si.at[b])
        pltpu.async_copy(cols_hbm.at[pl.ds(base, K)], col6.at[b], si.at[b])
        pltpu.async_copy(vals_hbm.at[pl.ds(base, K)], val6.at[b], si.at[b])

    def wait_idx(i, b):
        base = base_of(i)
        pltpu.make_async_copy(rows_hbm.at[pl.ds(base, K)], rowb[b], si.at[b]).wait()
        pltpu.make_async_copy(cols_hbm.at[pl.ds(base, K)], col6.at[b], si.at[b]).wait()
        pltpu.make_async_copy(vals_hbm.at[pl.ds(base, K)], val6.at[b], si.at[b]).wait()

    def start_gather(b):
        pltpu.async_copy(x_hbm.at[col6.at[b]], gath6.at[b], sg.at[b])

    def wait_gather(b):
        pltpu.make_async_copy(x_hbm.at[col6.at[b]], gath6.at[b], sg.at[b]).wait()

    with jax.named_scope("zph"):
        zsl = pl.ds(s * ROWS_T, ROWS_T)
        pltpu.sync_copy(zeros_hbm.at[zsl], acc.at[zsl])

        @pl.when(s == NS - 1)
        def _():
            zrem = pl.ds(NS * ROWS_T, REM_ROWS)
            pltpu.sync_copy(zeros_hbm.at[zrem], acc.at[zrem])

    with jax.named_scope("zbar"):
        plsc.subcore_barrier()

    def scale_chunk(gref, vref):
        himask = jnp.full((16,), -65536, jnp.int32)

        def grp(g, carry2):
            for jj in range(16):
                j = g * 16 + jj
                sp = plsc.load_gather(vref, [jnp.full((16,), j, jnp.int32)])
                r = gref.at[j]
                o = gathf.at[j]
                for f in range(D_FEAT // 32):
                    w = r[pl.ds(f * 16, 16)]
                    a = plsc.bitcast(w << 16, jnp.float32)
                    b2 = plsc.bitcast(w & himask, jnp.float32)
                    o[pl.ds(f * 32, 16)] = a * sp
                    o[pl.ds(f * 32 + 16, 16)] = b2 * sp
            return carry2

        lax.fori_loop(0, K // 16, grp, 0)

    def do_chunk(i, b):
        with jax.named_scope("gwait"):
            wait_gather(b)

        with jax.named_scope("iwait"):
            @pl.when(i + DEPTH < n_chunks)
            def _():
                bg = (b + DEPTH) % NB
                wait_idx(i + DEPTH, bg)
                start_gather(bg)

            @pl.when(i + DEPTH + 1 < n_chunks)
            def _():
                start_idx(i + DEPTH + 1, (b + DEPTH + 1) % NB)

        with jax.named_scope("scale"):
            scale_chunk(gath6.at[b], val6.at[b])
        with jax.named_scope("scat"):
            pltpu.sync_copy(gathf, acc.at[rowb[b]], add=True)

    for j in range(DEPTH + 1):
        start_idx(j, j)
    for j in range(DEPTH):
        wait_idx(j, j)
        start_gather(j)

    def five(k, carry):
        i0 = k * NB
        for off in range(NB):
            do_chunk(i0 + off, off)
        return carry

    lax.fori_loop(0, n_chunks // NB, five, 0)
    with jax.named_scope("ebar"):
        plsc.subcore_barrier()

    sl = pl.ds(s * ROWS_T, ROWS_T)
    rem = pl.ds(NS * ROWS_T, REM_ROWS)

    with jax.named_scope("wph"):
        @pl.when(c == 0)
        def _():
            pltpu.sync_copy(acc.at[sl], out0.at[sl])

            @pl.when(s == NS - 1)
            def _():
                pltpu.sync_copy(acc.at[rem], out0.at[rem])

        @pl.when(c == 1)
        def _():
            pltpu.sync_copy(acc.at[sl], out1.at[sl])

            @pl.when(s == NS - 1)
            def _():
                pltpu.sync_copy(acc.at[rem], out1.at[rem])


def _ew_call(body, n_out):
    out = tuple(jax.ShapeDtypeStruct((N_NODES, D_FEAT), jnp.float32)
                for _ in range(n_out))
    return pl.pallas_call(body, out_shape=out[0] if n_out == 1 else out)


def _add2_body(a, b, o):
    o[...] = a[...] + b[...]


def _resid_body(q0, q1, xp, tp, xo, to):
    x = q0[...] + q1[...] + xp[...]
    xo[...] = x
    to[...] = tp[...] + x


def _final_body(q0, q1, xp, tp, o):
    o[...] = (tp[...] + q0[...] + q1[...] + xp[...]) * 0.25


_add2 = _ew_call(_add2_body, 1)
_resid = _ew_call(_resid_body, 2)
_final = _ew_call(_final_body, 1)


def _to_bf16i(x):
    xi = x.reshape(N_NODES, D_FEAT // 32, 2, 16).swapaxes(2, 3)
    xb = xi.reshape(N_NODES, D_FEAT // 2, 2).astype(jnp.bfloat16)
    return lax.bitcast_convert_type(xb, jnp.int32)


def _prep(indices, values):
    idx = indices.astype(jnp.int32)
    pad = NNZ_PAD - NNZ
    rows = jnp.concatenate([idx[0], jnp.zeros((pad,), jnp.int32)])
    cols = jnp.concatenate([idx[1], jnp.zeros((pad,), jnp.int32)])
    vals = jnp.concatenate([values.astype(jnp.float32),
                            jnp.zeros((pad,), jnp.float32)])
    return rows, cols, vals


def kernel(poi_embs, src_indices, src_values, tar_indices, tar_values):
    tr, tcol, tval = _prep(tar_indices, tar_values)
    sr, scol, sval = _prep(src_indices, src_values)
    x = poi_embs
    t = poi_embs
    out = None
    zeros = jnp.zeros((N_NODES, D_FEAT), jnp.float32)
    for layer in range(3):
        p0, p1 = _spmm(_to_bf16i(x), tr, tcol, tval, zeros)
        m = _add2(p0, p1)
        q0, q1 = _spmm(_to_bf16i(m), sr, scol, sval, zeros)
        if layer < 2:
            x, t = _resid(q0, q1, x, t)
        else:
            out = _final(q0, q1, x, t)
    return out
